# baseline (speedup 1.0000x reference)
import functools

import jax
import jax.numpy as jnp
from jax import lax
from jax.experimental import pallas as pl
from jax.experimental.pallas import tpu as pltpu
from jax.experimental.pallas import tpu_sc as plsc

_LANES = 16
_PACK = 8


def kernel(word, table0, table1):
    B = word.shape[0]
    D = table0.shape[1]
    V = table0.shape[0]
    info = plsc.get_sparse_core_info()
    nw = info.num_cores * info.num_subcores
    b_per_w = B // nw
    half = b_per_w // 2
    g_per_w = b_per_w // _PACK

    t0v = table0.reshape(V // _PACK, _PACK * D)
    t1v = table1.reshape(V // _PACK, _PACK * D)

    mesh = plsc.VectorSubcoreMesh(core_axis_name="c", subcore_axis_name="s")

    @functools.partial(
        pl.kernel,
        mesh=mesh,
        out_type=jax.ShapeDtypeStruct((2 * B // _PACK, _PACK * D), jnp.float32),
        scratch_types=[
            pltpu.VMEM((b_per_w,), jnp.int32),
            pltpu.VMEM((half,), jnp.int32),
            pltpu.VMEM((half,), jnp.int32),
            pltpu.VMEM((half, _PACK * D), jnp.float32),
            pltpu.VMEM((half, _PACK * D), jnp.float32),
            pltpu.VMEM((g_per_w, _PACK * D), jnp.float32),
            pltpu.VMEM((g_per_w, _PACK * D), jnp.float32),
            pltpu.SemaphoreType.DMA,
            pltpu.SemaphoreType.DMA,
        ],
        compiler_params=pltpu.CompilerParams(needs_layout_passes=False),
    )
    def gather_kernel(word_hbm, t0_hbm, t1_hbm, out_hbm,
                      idx_v, d0_v, d1_v, rows_a, rows_b, out0_v, out1_v,
                      sem_a, sem_b):
        wid = lax.axis_index("s") * info.num_cores + lax.axis_index("c")
        base = wid * b_per_w
        pltpu.sync_copy(word_hbm.at[pl.ds(base, b_per_w)], idx_v)

        for k in range(b_per_w // _LANES):
            chunk = idx_v[pl.ds(k * _LANES, _LANES)] >> 3
            dst = d0_v if k < half // _LANES else d1_v
            dst[pl.ds((k * _LANES) % half, _LANES)] = chunk

        iota = lax.iota(jnp.int32, _LANES)
        ocol0 = (iota & (_PACK - 1)) * D

        def extract(rows_ref, out_ref, cb):
            def body(jb, _):
                j0 = cb + jb * _LANES
                lrow = jb * _LANES + iota
                offs = (idx_v[pl.ds(j0, _LANES)] & (_PACK - 1)) * D
                orow = j0 // _PACK + (iota >> 3)
                for c in range(D):
                    v = plsc.load_gather(rows_ref, [lrow, offs + c])
                    plsc.store_scatter(out_ref, [orow, ocol0 + c], v)
                return 0
            lax.fori_loop(0, half // _LANES, body, 0)

        c0 = pltpu.async_copy(t0_hbm.at[d0_v], rows_a, sem_a)
        c1 = pltpu.async_copy(t0_hbm.at[d1_v], rows_b, sem_b)
        c0.wait()
        extract(rows_a, out0_v, 0)
        c2 = pltpu.async_copy(t1_hbm.at[d0_v], rows_a, sem_a)
        c1.wait()
        extract(rows_b, out0_v, half)
        c3 = pltpu.async_copy(t1_hbm.at[d1_v], rows_b, sem_b)
        pltpu.sync_copy(out0_v, out_hbm.at[pl.ds(wid * g_per_w, g_per_w)])
        c2.wait()
        extract(rows_a, out1_v, 0)
        c3.wait()
        extract(rows_b, out1_v, half)
        pltpu.sync_copy(
            out1_v, out_hbm.at[pl.ds(B // _PACK + wid * g_per_w, g_per_w)])

    out = gather_kernel(word.astype(jnp.int32), t0v, t1v)
    return out.reshape(2 * B, D)


# speedup vs baseline: 4.6518x; 4.6518x over previous
import functools

import jax
import jax.numpy as jnp
from jax import lax
from jax.experimental import pallas as pl
from jax.experimental.pallas import tpu as pltpu
from jax.experimental.pallas import tpu_sc as plsc

_LANES = 16
_GRP = 16


def kernel(word, table0, table1):
    B = word.shape[0]
    V, D = table0.shape
    info = plsc.get_sparse_core_info()
    nw = info.num_cores * info.num_subcores
    b_per_w = B // nw
    max_off = ((V - 128) // 128) * 128

    tt0 = table0.T
    tt1 = table1.T
    tl0 = table0[V - 128:, :].T
    tl1 = table1[V - 128:, :].T

    mesh = plsc.VectorSubcoreMesh(core_axis_name="c", subcore_axis_name="s")

    @functools.partial(
        pl.kernel,
        mesh=mesh,
        out_type=jax.ShapeDtypeStruct((D, 2 * B), jnp.float32),
        scratch_types=[
            pltpu.VMEM((b_per_w,), jnp.int32),
            [pltpu.VMEM((D, 128), jnp.float32) for _ in range(_GRP)],
            [pltpu.VMEM((D, 128), jnp.float32) for _ in range(2)],
            pltpu.VMEM((D, b_per_w), jnp.float32),
            pltpu.VMEM((D, b_per_w), jnp.float32),
            pltpu.SemaphoreType.DMA,
        ],
        compiler_params=pltpu.CompilerParams(needs_layout_passes=False),
    )
    def gather_kernel(word_hbm, t0_hbm, t1_hbm, tl0_hbm, tl1_hbm, out_hbm,
                      idx_v, slabs, tails, out0_v, out1_v, sem):
        wid = lax.axis_index("s") * info.num_cores + lax.axis_index("c")
        base = wid * b_per_w
        pltpu.sync_copy(word_hbm.at[pl.ds(base, b_per_w)], idx_v)
        pltpu.sync_copy(tl0_hbm, tails[0])
        pltpu.sync_copy(tl1_hbm, tails[1])

        iota = lax.iota(jnp.int32, _LANES)

        def do_group(t_hbm, tail_v, out_v, g):
            vec = idx_v[pl.ds(g, _GRP)]
            scalars = []
            copies = []
            for l in range(_GRP):
                w = jnp.max(jnp.where(iota == l, vec, jnp.int32(0)))
                off = jnp.minimum((w >> 7) * 128, jnp.int32(max_off))
                off = pl.multiple_of(off, 128)
                scalars.append((w, off))
                copies.append(
                    pltpu.async_copy(
                        t_hbm.at[:, pl.ds(off, 128)], slabs[l], sem))
            for l in range(_GRP):
                copies[l].wait()
            for l in range(_GRP):
                w, off = scalars[l]
                wl = jnp.broadcast_to(jnp.minimum(w - off, 127), (_LANES,))
                col = plsc.load_gather(slabs[l], [iota, wl])
                wt = jnp.broadcast_to(
                    jnp.clip(w - (V - 128), 0, 127), (_LANES,))
                col_t = plsc.load_gather(tail_v, [iota, wt])
                is_tail = jnp.broadcast_to(w >= V - 64, (_LANES,))
                col = jnp.where(is_tail, col_t, col)
                jcol = jnp.broadcast_to(g + l, (_LANES,))
                plsc.store_scatter(out_v, [iota, jcol], col)

        def body(k, _):
            do_group(t0_hbm, tails[0], out0_v, k * _GRP)
            do_group(t1_hbm, tails[1], out1_v, k * _GRP)
            return 0
        lax.fori_loop(0, b_per_w // _GRP, body, 0)

        pltpu.sync_copy(out0_v, out_hbm.at[:, pl.ds(base, b_per_w)])
        pltpu.sync_copy(out1_v, out_hbm.at[:, pl.ds(B + base, b_per_w)])

    out_t = gather_kernel(word.astype(jnp.int32), tt0, tt1, tl0, tl1)
    return out_t.T


# speedup vs baseline: 5.8971x; 1.2677x over previous
import functools

import jax
import jax.numpy as jnp
from jax import lax
from jax.experimental import pallas as pl
from jax.experimental.pallas import tpu as pltpu
from jax.experimental.pallas import tpu_sc as plsc

_LANES = 16
_GRP = 16


def kernel(word, table0, table1):
    B = word.shape[0]
    V, D = table0.shape
    info = plsc.get_sparse_core_info()
    nw = info.num_cores * info.num_subcores
    b_per_w = B // nw
    max_off = ((V - 128) // 128) * 128

    tt0 = table0.T
    tt1 = table1.T
    tl0 = table0[V - 128:, :].T
    tl1 = table1[V - 128:, :].T

    mesh = plsc.VectorSubcoreMesh(core_axis_name="c", subcore_axis_name="s")

    @functools.partial(
        pl.kernel,
        mesh=mesh,
        out_type=jax.ShapeDtypeStruct((D, 2 * B), jnp.float32),
        scratch_types=[
            pltpu.VMEM((b_per_w,), jnp.int32),
            [pltpu.VMEM((D, 128), jnp.float32) for _ in range(_GRP)],
            [pltpu.VMEM((D, 128), jnp.float32) for _ in range(_GRP)],
            [pltpu.VMEM((D, 128), jnp.float32) for _ in range(2)],
            pltpu.VMEM((D, b_per_w), jnp.float32),
            pltpu.VMEM((D, b_per_w), jnp.float32),
            pltpu.SemaphoreType.DMA,
            pltpu.SemaphoreType.DMA,
        ],
        compiler_params=pltpu.CompilerParams(needs_layout_passes=False),
    )
    def gather_kernel(word_hbm, t0_hbm, t1_hbm, tl0_hbm, tl1_hbm, out_hbm,
                      idx_v, bank_a, bank_b, tails, out0_v, out1_v,
                      sem_a, sem_b):
        wid = lax.axis_index("s") * info.num_cores + lax.axis_index("c")
        base = wid * b_per_w
        pltpu.sync_copy(word_hbm.at[pl.ds(base, b_per_w)], idx_v)
        pltpu.sync_copy(tl0_hbm, tails[0])
        pltpu.sync_copy(tl1_hbm, tails[1])

        iota = lax.iota(jnp.int32, _LANES)
        n_grp = b_per_w // _GRP

        def offsets(g):
            vec = idx_v[pl.ds(g, _GRP)]
            out = []
            for l in range(_GRP):
                w = jnp.max(jnp.where(iota == l, vec, jnp.int32(0)))
                off = jnp.minimum((w >> 7) * 128, jnp.int32(max_off))
                out.append((w, pl.multiple_of(off, 128)))
            return out

        def fire(t_hbm, bank, sem, g):
            for l, (_, off) in enumerate(offsets(g)):
                pltpu.async_copy(t_hbm.at[:, pl.ds(off, 128)], bank[l], sem)

        def drain_extract(t_hbm, bank, sem, tail_v, out_v, g):
            for l in range(_GRP):
                pltpu.make_async_copy(
                    t_hbm.at[:, pl.ds(0, 128)], bank[l], sem).wait()
            for l, (w, off) in enumerate(offsets(g)):
                wl = jnp.broadcast_to(jnp.minimum(w - off, 127), (_LANES,))
                col = plsc.load_gather(bank[l], [iota, wl])
                wt = jnp.broadcast_to(
                    jnp.clip(w - (V - 128), 0, 127), (_LANES,))
                col_t = plsc.load_gather(tail_v, [iota, wt])
                is_tail = jnp.broadcast_to(w >= V - 64, (_LANES,))
                col = jnp.where(is_tail, col_t, col)
                jcol = jnp.broadcast_to(g + l, (_LANES,))
                plsc.store_scatter(out_v, [iota, jcol], col)

        fire(t0_hbm, bank_a, sem_a, 0)

        def body(k, _):
            g = k * _GRP
            fire(t1_hbm, bank_b, sem_b, g)
            drain_extract(t0_hbm, bank_a, sem_a, tails[0], out0_v, g)

            @pl.when(k < n_grp - 1)
            def _():
                fire(t0_hbm, bank_a, sem_a, g + _GRP)

            drain_extract(t1_hbm, bank_b, sem_b, tails[1], out1_v, g)
            return 0
        lax.fori_loop(0, n_grp, body, 0)

        pltpu.sync_copy(out0_v, out_hbm.at[:, pl.ds(base, b_per_w)])
        pltpu.sync_copy(out1_v, out_hbm.at[:, pl.ds(B + base, b_per_w)])

    out_t = gather_kernel(word.astype(jnp.int32), tt0, tt1, tl0, tl1)
    return out_t.T
